# baseline (speedup 1.0000x reference)
import functools

import numpy as np
import jax
import jax.numpy as jnp
from jax import lax
from jax.experimental import pallas as pl
from jax.experimental.pallas import tpu as pltpu
from jax.experimental.pallas import tpu_sc as plsc

_N = 8192
_IN = 4096
_RED = 1024
_K = 2048
_LANES = 16
_NV = _N // _LANES


_BLK = 512


def _score_body(x_ref, w1t_ref, b1_ref, w2t_ref, b2_ref, g_ref, s_ref):
  z1 = jnp.dot(x_ref[...], w1t_ref[...], preferred_element_type=jnp.float32)
  z1 = jnp.maximum(z1 + b1_ref[...], 0.0)
  z2 = jnp.dot(z1, w2t_ref[...], preferred_element_type=jnp.float32)
  z2 = z2 + b2_ref[...]
  s = 1.0 / (1.0 + jnp.exp(-z2))
  s_ref[...] = s.reshape(_BLK) + g_ref[...]


def _scores(x, w1t, b1r, w2t, b2r, gum):
  return pl.pallas_call(
      _score_body,
      grid=(_N // _BLK,),
      in_specs=[
          pl.BlockSpec((_BLK, _IN), lambda i: (i, 0)),
          pl.BlockSpec((_IN, _RED), lambda i: (0, 0)),
          pl.BlockSpec((1, _RED), lambda i: (0, 0)),
          pl.BlockSpec((_RED, 1), lambda i: (0, 0)),
          pl.BlockSpec((1, 1), lambda i: (0, 0)),
          pl.BlockSpec((_BLK,), lambda i: (i,)),
      ],
      out_specs=pl.BlockSpec((_BLK,), lambda i: (i,)),
      out_shape=jax.ShapeDtypeStruct((_N,), jnp.float32),
  )(x, w1t, b1r, w2t, b2r, gum)


_IMIN = np.int32(-2147483648)
_IMAXP = np.int32(2147483647)
_NBINS = 256


def _gate_sc_body(s_hbm, gate_hbm, s_v, key_v, gate_v, hist_v):
  cid = lax.axis_index("c")
  sid = lax.axis_index("s")
  is_worker = jnp.logical_and(cid == 0, sid == 0)

  @pl.when(is_worker)
  def _():
    pltpu.sync_copy(s_hbm, s_v)

    def mk_body(i, _):
      u = plsc.bitcast(s_v[pl.ds(i * _LANES, _LANES)], jnp.int32)
      m = lax.shift_right_arithmetic(u, 31)
      key_v[pl.ds(i * _LANES, _LANES)] = u ^ (m & _IMAXP)
      return 0

    lax.fori_loop(0, _NV, mk_body, 0, unroll=8)

    ones = jnp.ones((_LANES,), jnp.int32)

    prefix = jnp.int32(0)
    n_gt = jnp.int32(0)
    for lvl in range(4):
      shift = 24 - 8 * lvl

      def zero_body(i, _):
        hist_v[pl.ds(i * _LANES, _LANES)] = jnp.zeros((_LANES,), jnp.int32)
        return 0

      lax.fori_loop(0, _NBINS // _LANES, zero_body, 0, unroll=4)

      def hist_body(i, _, shift=shift, prefix=prefix):
        ku = key_v[pl.ds(i * _LANES, _LANES)] ^ _IMIN
        byte = lax.shift_right_logical(ku, shift) & jnp.int32(0xFF)
        if shift == 24:
          plsc.addupdate_scatter(hist_v, [byte], ones)
        else:
          sel = lax.shift_right_logical(ku, shift + 8) == prefix
          plsc.addupdate_scatter(hist_v, [byte], ones, mask=sel)
        return 0

      lax.fori_loop(0, _NV, hist_body, 0, unroll=4)

      need = jnp.int32(_K) - n_gt
      cum = jnp.int32(0)
      b_sel = jnp.int32(-1)
      n_above = jnp.int32(0)
      lane = lax.iota(jnp.int32, _LANES)
      for j in range(_NBINS // _LANES - 1, -1, -1):
        v = hist_v[pl.ds(j * _LANES, _LANES)]
        rv = lax.rev(v, (0,))
        pc = plsc.cumsum(rv)
        hit = (cum + pc) >= need
        bin_vec = jnp.int32(j * _LANES + _LANES - 1) - lane
        b_here = jnp.max(jnp.where(hit, bin_vec, jnp.int32(-1)))
        above_here = cum + jnp.sum(jnp.where(hit, jnp.int32(0), rv))
        found_now = jnp.logical_and(b_sel < 0, b_here >= 0)
        b_sel = jnp.where(found_now, b_here, b_sel)
        n_above = jnp.where(found_now, above_here, n_above)
        cum = cum + jnp.sum(rv)

      n_gt = n_gt + n_above
      prefix = (prefix << 8) | b_sel

    thr = prefix ^ _IMIN
    r_ties = jnp.int32(_K) - n_gt

    def gate_body(i, run):
      kv = key_v[pl.ds(i * _LANES, _LANES)]
      gt = kv > thr
      eq = kv == thr
      eqi = jnp.where(eq, jnp.int32(1), jnp.int32(0))
      incl = plsc.cumsum(eqi)
      rank = run + (incl - eqi)
      sel = jnp.logical_or(gt, jnp.logical_and(eq, rank < r_ties))
      gate_v[pl.ds(i * _LANES, _LANES)] = jnp.where(sel, 1.0, 0.0).astype(
          jnp.float32)
      return run + incl[_LANES - 1]

    lax.fori_loop(0, _NV, gate_body, jnp.int32(0), unroll=4)

    pltpu.sync_copy(gate_v, gate_hbm)


@functools.cache
def _gate_sc_kernel():
  return pl.kernel(
      _gate_sc_body,
      out_type=jax.ShapeDtypeStruct((_N,), jnp.float32),
      mesh=plsc.VectorSubcoreMesh(core_axis_name="c", subcore_axis_name="s"),
      scratch_types=[
          pltpu.VMEM((_N,), jnp.float32),
          pltpu.VMEM((_N,), jnp.int32),
          pltpu.VMEM((_N,), jnp.float32),
          pltpu.VMEM((_NBINS,), jnp.int32),
      ],
      compiler_params=pltpu.CompilerParams(
          needs_layout_passes=False, use_tc_tiling_on_sc=True),
  )


def _mask_body(x_ref, g_ref, o_ref):
  g = g_ref[...].reshape(_BLK, 1)
  o_ref[...] = x_ref[...] * g


def _mask(x, gate):
  return pl.pallas_call(
      _mask_body,
      grid=(_N // _BLK,),
      in_specs=[
          pl.BlockSpec((_BLK, _IN), lambda i: (i, 0)),
          pl.BlockSpec((_BLK,), lambda i: (i,)),
      ],
      out_specs=pl.BlockSpec((_BLK, _IN), lambda i: (i, 0)),
      out_shape=jax.ShapeDtypeStruct((_N, _IN), jnp.float32),
  )(x, gate)


@jax.jit
def kernel(x, W1, b1, W2, b2, gumbels):
  w1t = W1.T
  b1r = b1.reshape(1, _RED)
  w2t = W2.T
  b2r = b2.reshape(1, 1)
  s = _scores(x, w1t, b1r, w2t, b2r, gumbels)
  gate = _gate_sc_kernel()(s)
  return _mask(x, gate)


# speedup vs baseline: 1.1560x; 1.1560x over previous
import functools

import numpy as np
import jax
import jax.numpy as jnp
from jax import lax
from jax.experimental import pallas as pl
from jax.experimental.pallas import tpu as pltpu
from jax.experimental.pallas import tpu_sc as plsc

_N = 8192
_IN = 4096
_RED = 1024
_K = 2048
_LANES = 16
_NV = _N // _LANES


_BLK = 512


def _score_body(x_ref, w1t_ref, b1_ref, w2t_ref, b2_ref, g_ref, s_ref):
  z1 = jnp.dot(x_ref[...], w1t_ref[...], preferred_element_type=jnp.float32)
  z1 = jnp.maximum(z1 + b1_ref[...], 0.0)
  z2 = jnp.dot(z1, w2t_ref[...], preferred_element_type=jnp.float32)
  z2 = z2 + b2_ref[...]
  s = 1.0 / (1.0 + jnp.exp(-z2))
  s_ref[...] = s.reshape(_BLK) + g_ref[...]


def _scores(x, w1t, b1r, w2t, b2r, gum):
  return pl.pallas_call(
      _score_body,
      grid=(_N // _BLK,),
      in_specs=[
          pl.BlockSpec((_BLK, _IN), lambda i: (i, 0)),
          pl.BlockSpec((_IN, _RED), lambda i: (0, 0)),
          pl.BlockSpec((1, _RED), lambda i: (0, 0)),
          pl.BlockSpec((_RED, 1), lambda i: (0, 0)),
          pl.BlockSpec((1, 1), lambda i: (0, 0)),
          pl.BlockSpec((_BLK,), lambda i: (i,)),
      ],
      out_specs=pl.BlockSpec((_BLK,), lambda i: (i,)),
      out_shape=jax.ShapeDtypeStruct((_N,), jnp.float32),
  )(x, w1t, b1r, w2t, b2r, gum)


_IMIN = np.int32(-2147483648)
_IMAXP = np.int32(2147483647)
_NBINS = 256


def _gate_sc_body(s_hbm, gate_hbm, s_v, key_v, gate_v, hist_v):
  cid = lax.axis_index("c")
  sid = lax.axis_index("s")
  is_worker = jnp.logical_and(cid == 0, sid == 0)

  @pl.when(is_worker)
  def _():
    pltpu.sync_copy(s_hbm, s_v)

    def mk_body(i, _):
      u = plsc.bitcast(s_v[pl.ds(i * _LANES, _LANES)], jnp.int32)
      m = lax.shift_right_arithmetic(u, 31)
      key_v[pl.ds(i * _LANES, _LANES)] = u ^ (m & _IMAXP)
      return 0

    lax.fori_loop(0, _NV, mk_body, 0, unroll=8)

    ones = jnp.ones((_LANES,), jnp.int32)

    prefix = jnp.int32(0)
    n_gt = jnp.int32(0)
    for lvl in range(4):
      shift = 24 - 8 * lvl

      def zero_body(i, _):
        hist_v[pl.ds(i * _LANES, _LANES)] = jnp.zeros((_LANES,), jnp.int32)
        return 0

      lax.fori_loop(0, _NBINS // _LANES, zero_body, 0, unroll=4)

      def hist_body(i, _, shift=shift, prefix=prefix):
        ku = key_v[pl.ds(i * _LANES, _LANES)] ^ _IMIN
        byte = lax.shift_right_logical(ku, shift) & jnp.int32(0xFF)
        if shift == 24:
          plsc.addupdate_scatter(hist_v, [byte], ones)
        else:
          sel = lax.shift_right_logical(ku, shift + 8) == prefix
          plsc.addupdate_scatter(hist_v, [byte], ones, mask=sel)
        return 0

      lax.fori_loop(0, _NV, hist_body, 0, unroll=4)

      need = jnp.int32(_K) - n_gt
      cum = jnp.int32(0)
      b_sel = jnp.int32(-1)
      n_above = jnp.int32(0)
      lane = lax.iota(jnp.int32, _LANES)
      for j in range(_NBINS // _LANES - 1, -1, -1):
        v = hist_v[pl.ds(j * _LANES, _LANES)]
        rv = lax.rev(v, (0,))
        pc = plsc.cumsum(rv)
        hit = (cum + pc) >= need
        bin_vec = jnp.int32(j * _LANES + _LANES - 1) - lane
        b_here = jnp.max(jnp.where(hit, bin_vec, jnp.int32(-1)))
        above_here = cum + jnp.sum(jnp.where(hit, jnp.int32(0), rv))
        found_now = jnp.logical_and(b_sel < 0, b_here >= 0)
        b_sel = jnp.where(found_now, b_here, b_sel)
        n_above = jnp.where(found_now, above_here, n_above)
        cum = cum + jnp.sum(rv)

      n_gt = n_gt + n_above
      prefix = (prefix << 8) | b_sel

    thr = prefix ^ _IMIN
    r_ties = jnp.int32(_K) - n_gt

    def gate_body(i, run):
      kv = key_v[pl.ds(i * _LANES, _LANES)]
      gt = kv > thr
      eq = kv == thr
      eqi = jnp.where(eq, jnp.int32(1), jnp.int32(0))
      incl = plsc.cumsum(eqi)
      rank = run + (incl - eqi)
      sel = jnp.logical_or(gt, jnp.logical_and(eq, rank < r_ties))
      gate_v[pl.ds(i * _LANES, _LANES)] = jnp.where(sel, 1.0, 0.0).astype(
          jnp.float32)
      return run + incl[_LANES - 1]

    lax.fori_loop(0, _NV, gate_body, jnp.int32(0), unroll=4)

    pltpu.sync_copy(gate_v, gate_hbm)


@functools.cache
def _gate_sc_kernel():
  return pl.kernel(
      _gate_sc_body,
      out_type=jax.ShapeDtypeStruct((_N,), jnp.float32),
      mesh=plsc.VectorSubcoreMesh(core_axis_name="c", subcore_axis_name="s"),
      scratch_types=[
          pltpu.VMEM((_N,), jnp.float32),
          pltpu.VMEM((_N,), jnp.int32),
          pltpu.VMEM((_N,), jnp.float32),
          pltpu.VMEM((_NBINS,), jnp.int32),
      ],
      compiler_params=pltpu.CompilerParams(
          needs_layout_passes=False, use_tc_tiling_on_sc=True),
  )


def _mask_body(x_ref, g_ref, o_ref):
  g = g_ref[...].reshape(_BLK, 1)
  o_ref[...] = x_ref[...] * g


def _mask(x, gate):
  return pl.pallas_call(
      _mask_body,
      grid=(_N // _BLK,),
      in_specs=[
          pl.BlockSpec((_BLK, _IN), lambda i: (i, 0)),
          pl.BlockSpec((_BLK,), lambda i: (i,)),
      ],
      out_specs=pl.BlockSpec((_BLK, _IN), lambda i: (i, 0)),
      out_shape=jax.ShapeDtypeStruct((_N, _IN), jnp.float32),
  )(x, gate)


@jax.jit
def kernel(x, W1, b1, W2, b2, gumbels):
  w1t = W1.T
  b1r = b1.reshape(1, _RED)
  w2t = W2.T
  b2r = b2.reshape(1, 1)
  s = _scores(x, w1t, b1r, w2t, b2r, gumbels)
  return _mask(x, s)


# speedup vs baseline: 2.9482x; 2.5503x over previous
import functools

import numpy as np
import jax
import jax.numpy as jnp
from jax import lax
from jax.experimental import pallas as pl
from jax.experimental.pallas import tpu as pltpu
from jax.experimental.pallas import tpu_sc as plsc

_N = 8192
_IN = 4096
_RED = 1024
_K = 2048
_LANES = 16
_NV = _N // _LANES


_BLK = 512


def _score_body(x_ref, w1t_ref, b1_ref, w2t_ref, b2_ref, g_ref, s_ref):
  z1 = jnp.dot(x_ref[...], w1t_ref[...], preferred_element_type=jnp.float32)
  z1 = jnp.maximum(z1 + b1_ref[...], 0.0)
  z2 = jnp.dot(z1, w2t_ref[...], preferred_element_type=jnp.float32)
  z2 = z2 + b2_ref[...]
  s = 1.0 / (1.0 + jnp.exp(-z2))
  s_ref[...] = s.reshape(_BLK) + g_ref[...]


def _scores(x, w1t, b1r, w2t, b2r, gum):
  return pl.pallas_call(
      _score_body,
      grid=(_N // _BLK,),
      in_specs=[
          pl.BlockSpec((_BLK, _IN), lambda i: (i, 0)),
          pl.BlockSpec((_IN, _RED), lambda i: (0, 0)),
          pl.BlockSpec((1, _RED), lambda i: (0, 0)),
          pl.BlockSpec((_RED, 1), lambda i: (0, 0)),
          pl.BlockSpec((1, 1), lambda i: (0, 0)),
          pl.BlockSpec((_BLK,), lambda i: (i,)),
      ],
      out_specs=pl.BlockSpec((_BLK,), lambda i: (i,)),
      out_shape=jax.ShapeDtypeStruct((_N,), jnp.float32),
  )(x, w1t, b1r, w2t, b2r, gum)


_IMIN = np.int32(-2147483648)
_IMAXP = np.int32(2147483647)
_NBINS = 256


def _gate_sc_body(s_hbm, gate_hbm, s_v, key_v, gate_v, hist_v):
  cid = lax.axis_index("c")
  sid = lax.axis_index("s")
  is_worker = jnp.logical_and(cid == 0, sid == 0)

  @pl.when(is_worker)
  def _():
    pltpu.sync_copy(s_hbm, s_v)

    def mk_body(i, _):
      u = plsc.bitcast(s_v[pl.ds(i * _LANES, _LANES)], jnp.int32)
      m = lax.shift_right_arithmetic(u, 31)
      key_v[pl.ds(i * _LANES, _LANES)] = u ^ (m & _IMAXP)
      return 0

    lax.fori_loop(0, _NV, mk_body, 0, unroll=8)

    ones = jnp.ones((_LANES,), jnp.int32)

    prefix = jnp.int32(0)
    n_gt = jnp.int32(0)
    for lvl in range(4):
      shift = 24 - 8 * lvl

      def zero_body(i, _):
        hist_v[pl.ds(i * _LANES, _LANES)] = jnp.zeros((_LANES,), jnp.int32)
        return 0

      lax.fori_loop(0, _NBINS // _LANES, zero_body, 0, unroll=4)

      def hist_body(i, _, shift=shift, prefix=prefix):
        ku = key_v[pl.ds(i * _LANES, _LANES)] ^ _IMIN
        byte = lax.shift_right_logical(ku, shift) & jnp.int32(0xFF)
        if shift == 24:
          plsc.addupdate_scatter(hist_v, [byte], ones)
        else:
          sel = lax.shift_right_logical(ku, shift + 8) == prefix
          plsc.addupdate_scatter(hist_v, [byte], ones, mask=sel)
        return 0

      lax.fori_loop(0, _NV, hist_body, 0, unroll=4)

      need = jnp.int32(_K) - n_gt
      cum = jnp.int32(0)
      b_sel = jnp.int32(-1)
      n_above = jnp.int32(0)
      lane = lax.iota(jnp.int32, _LANES)
      for j in range(_NBINS // _LANES - 1, -1, -1):
        v = hist_v[pl.ds(j * _LANES, _LANES)]
        rv = lax.rev(v, (0,))
        pc = plsc.cumsum(rv)
        hit = (cum + pc) >= need
        bin_vec = jnp.int32(j * _LANES + _LANES - 1) - lane
        b_here = jnp.max(jnp.where(hit, bin_vec, jnp.int32(-1)))
        above_here = cum + jnp.sum(jnp.where(hit, jnp.int32(0), rv))
        found_now = jnp.logical_and(b_sel < 0, b_here >= 0)
        b_sel = jnp.where(found_now, b_here, b_sel)
        n_above = jnp.where(found_now, above_here, n_above)
        cum = cum + jnp.sum(rv)

      n_gt = n_gt + n_above
      prefix = (prefix << 8) | b_sel

    thr = prefix ^ _IMIN
    r_ties = jnp.int32(_K) - n_gt

    def gate_body(i, run):
      kv = key_v[pl.ds(i * _LANES, _LANES)]
      gt = kv > thr
      eq = kv == thr
      eqi = jnp.where(eq, jnp.int32(1), jnp.int32(0))
      incl = plsc.cumsum(eqi)
      rank = run + (incl - eqi)
      sel = jnp.logical_or(gt, jnp.logical_and(eq, rank < r_ties))
      gate_v[pl.ds(i * _LANES, _LANES)] = jnp.where(sel, 1.0, 0.0).astype(
          jnp.float32)
      return run + incl[_LANES - 1]

    lax.fori_loop(0, _NV, gate_body, jnp.int32(0), unroll=4)

    pltpu.sync_copy(gate_v, gate_hbm)


@functools.cache
def _gate_sc_kernel():
  return pl.kernel(
      _gate_sc_body,
      out_type=jax.ShapeDtypeStruct((_N,), jnp.float32),
      mesh=plsc.VectorSubcoreMesh(core_axis_name="c", subcore_axis_name="s"),
      scratch_types=[
          pltpu.VMEM((_N,), jnp.float32),
          pltpu.VMEM((_N,), jnp.int32),
          pltpu.VMEM((_N,), jnp.float32),
          pltpu.VMEM((_NBINS,), jnp.int32),
      ],
      compiler_params=pltpu.CompilerParams(
          needs_layout_passes=False, use_tc_tiling_on_sc=True),
  )


def _mask_body(x_ref, g_ref, o_ref):
  g = g_ref[...].reshape(_BLK, 1)
  o_ref[...] = x_ref[...] * g


def _mask(x, gate):
  return pl.pallas_call(
      _mask_body,
      grid=(_N // _BLK,),
      in_specs=[
          pl.BlockSpec((_BLK, _IN), lambda i: (i, 0)),
          pl.BlockSpec((_BLK,), lambda i: (i,)),
      ],
      out_specs=pl.BlockSpec((_BLK, _IN), lambda i: (i, 0)),
      out_shape=jax.ShapeDtypeStruct((_N, _IN), jnp.float32),
  )(x, gate)


@jax.jit
def kernel(x, W1, b1, W2, b2, gumbels):
  w1t = W1.T
  b1r = b1.reshape(1, _RED)
  w2t = W2.T
  b2r = b2.reshape(1, 1)
  return _mask(x, gumbels)
